# Initial kernel scaffold; baseline (speedup 1.0000x reference)
import jax
import jax.numpy as jnp
from jax.experimental import pallas as pl


def kernel(indices, w_es, x):
    raise NotImplementedError("write your pallas kernel here")


# speedup vs baseline: 1.0790x; 1.0790x over previous
import functools

import jax
import jax.numpy as jnp
from jax import lax
from jax.experimental import pallas as pl
from jax.experimental.pallas import tpu as pltpu
from jax.experimental.pallas import tpu_sc as plsc

V_SIZE = 100000
B = 1024
L = 200
LP = 216
NC, NS = 2, 16
NW = NC * NS
ROWS = B // NW
NCH = (L + 15) // 16


def _softmax_body(w_ref, o_ref):
    w = w_ref[...]
    m = jnp.max(w, axis=-1, keepdims=True)
    e = jnp.exp(w - m)
    o_ref[...] = e / jnp.sum(e, axis=-1, keepdims=True)


_softmax = pl.pallas_call(
    _softmax_body,
    out_shape=jax.ShapeDtypeStruct((B, L), jnp.float32),
)

_mesh = plsc.VectorSubcoreMesh(core_axis_name="c", subcore_axis_name="s")


@functools.partial(
    pl.kernel,
    mesh=_mesh,
    compiler_params=pltpu.CompilerParams(
        needs_layout_passes=False, use_tc_tiling_on_sc=False
    ),
    out_type=jax.ShapeDtypeStruct((B, V_SIZE), jnp.float32),
    scratch_types=[
        pltpu.VMEM((V_SIZE + 16,), jnp.float32),
        pltpu.VMEM((LP,), jnp.int32),
        pltpu.VMEM((LP,), jnp.float32),
    ],
)
def _onehot_scatter(xs_hbm, ws_hbm, w_a_hbm, rowbuf, xv, wv):
    wid = lax.axis_index("s") * NC + lax.axis_index("c")
    zeros16 = jnp.zeros((16,), jnp.float32)

    def _memset(i, carry):
        rowbuf[pl.ds(i * 16, 16)] = zeros16
        return carry

    lax.fori_loop(0, (V_SIZE + 16) // 16, _memset, 0)
    xv[pl.ds(L, 16)] = jnp.full((16,), V_SIZE, jnp.int32)
    wv[pl.ds(L, 16)] = zeros16

    def _row(r, carry):
        b = wid * ROWS + r
        pltpu.sync_copy(xs_hbm.at[b], xv.at[pl.ds(0, L)])
        pltpu.sync_copy(ws_hbm.at[b], wv.at[pl.ds(0, L)])
        for k in range(NCH):
            plsc.store_scatter(
                rowbuf, [xv[pl.ds(k * 16, 16)]], wv[pl.ds(k * 16, 16)])
        pltpu.sync_copy(rowbuf.at[pl.ds(0, V_SIZE)], w_a_hbm.at[b])
        for k in range(NCH):
            plsc.store_scatter(rowbuf, [xv[pl.ds(k * 16, 16)]], zeros16)
        return carry

    lax.fori_loop(0, ROWS, _row, 0)


def kernel(indices, w_es, x):
    del indices
    x = x.astype(jnp.int32)
    weights = _softmax(w_es)

    b_idx = jnp.broadcast_to(jnp.arange(B, dtype=jnp.int32)[:, None], (B, L))
    key = (x * B + b_idx).reshape(-1)
    kg, wg = lax.sort_key_val(key, weights.reshape(-1), is_stable=False)
    nxt = jnp.concatenate([kg[1:], jnp.full((1,), -1, jnp.int32)])
    keep = kg != nxt
    bb = kg % B
    vv = kg // B
    key2 = bb * (V_SIZE + 1) + jnp.where(keep, vv, V_SIZE)
    kg2, wg2 = lax.sort_key_val(key2, wg, is_stable=False)
    xs = (kg2 % (V_SIZE + 1)).reshape(B, L)
    ws = wg2.reshape(B, L)

    w_a = _onehot_scatter(xs, ws)
    return (w_a, weights)


# speedup vs baseline: 1.1174x; 1.0355x over previous
import functools

import jax
import jax.numpy as jnp
from jax import lax
from jax.experimental import pallas as pl
from jax.experimental.pallas import tpu as pltpu
from jax.experimental.pallas import tpu_sc as plsc

V_SIZE = 100000
B = 1024
L = 200
LP = 216
NC, NS = 2, 16
NW = NC * NS
ROWS = B // NW
NCH = (L + 15) // 16


def _softmax_body(w_ref, o_ref):
    w = w_ref[...]
    m = jnp.max(w, axis=-1, keepdims=True)
    e = jnp.exp(w - m)
    o_ref[...] = e / jnp.sum(e, axis=-1, keepdims=True)


_softmax = pl.pallas_call(
    _softmax_body,
    out_shape=jax.ShapeDtypeStruct((B, L), jnp.float32),
)

_mesh = plsc.VectorSubcoreMesh(core_axis_name="c", subcore_axis_name="s")


HALF = V_SIZE // 2


@functools.partial(
    pl.kernel,
    mesh=_mesh,
    compiler_params=pltpu.CompilerParams(
        needs_layout_passes=False, use_tc_tiling_on_sc=False
    ),
    out_type=jax.ShapeDtypeStruct((B, V_SIZE), jnp.float32),
    scratch_types=[
        pltpu.VMEM((HALF,), jnp.float32),
        pltpu.VMEM((HALF,), jnp.float32),
        pltpu.VMEM((ROWS, L), jnp.int32),
        pltpu.VMEM((ROWS, L), jnp.float32),
        pltpu.SemaphoreType.DMA,
        pltpu.SemaphoreType.DMA,
    ],
)
def _onehot_scatter(xs_hbm, ws_hbm, w_a_hbm, bufa, bufb, xv, wv, sema, semb):
    wid = lax.axis_index("s") * NC + lax.axis_index("c")
    zeros16 = jnp.zeros((16,), jnp.float32)

    def _memset(i, carry):
        bufa[pl.ds(i * 16, 16)] = zeros16
        bufb[pl.ds(i * 16, 16)] = zeros16
        return carry

    lax.fori_loop(0, HALF // 16, _memset, 0)
    pltpu.sync_copy(xs_hbm.at[pl.ds(wid * ROWS, ROWS)], xv)
    pltpu.sync_copy(ws_hbm.at[pl.ds(wid * ROWS, ROWS)], wv)

    def _row(r, carry):
        b = wid * ROWS + r
        idxs = []
        vals = []
        for k in range(NCH - 1):
            idxs.append(xv[r, pl.ds(k * 16, 16)])
            vals.append(wv[r, pl.ds(k * 16, 16)])
        idxs.append(xv[r, pl.ds(L - 16, 16)])
        vals.append(wv[r, pl.ds(L - 16, 16)])

        def scatter_half(buf, lo, restore):
            for k in range(NCH):
                idx = idxs[k] - lo
                mask = (idx >= 0) & (idx < HALF)
                plsc.store_scatter(
                    buf, [jnp.where(mask, idx, 0)],
                    zeros16 if restore else vals[k], mask=mask)

        scatter_half(bufa, 0, False)
        cpa = pltpu.async_copy(bufa, w_a_hbm.at[b, pl.ds(0, HALF)], sema)
        scatter_half(bufb, HALF, False)
        cpb = pltpu.async_copy(bufb, w_a_hbm.at[b, pl.ds(HALF, HALF)], semb)
        cpa.wait()
        scatter_half(bufa, 0, True)
        cpb.wait()
        scatter_half(bufb, HALF, True)
        return carry

    lax.fori_loop(0, ROWS, _row, 0)


def kernel(indices, w_es, x):
    del indices
    x = x.astype(jnp.int32)
    weights = _softmax(w_es)

    b_idx = jnp.broadcast_to(jnp.arange(B, dtype=jnp.int32)[:, None], (B, L))
    key = (x * B + b_idx).reshape(-1)
    kg, wg = lax.sort_key_val(key, weights.reshape(-1), is_stable=False)
    nxt = jnp.concatenate([kg[1:], jnp.full((1,), -1, jnp.int32)])
    keep = kg != nxt
    bb = kg % B
    vv = kg // B
    key2 = bb * (V_SIZE + 1) + jnp.where(keep, vv, V_SIZE)
    kg2, wg2 = lax.sort_key_val(key2, wg, is_stable=False)
    xs = (kg2 % (V_SIZE + 1)).reshape(B, L)
    ws = wg2.reshape(B, L)

    w_a = _onehot_scatter(xs, ws)
    return (w_a, weights)
